# Initial kernel scaffold; baseline (speedup 1.0000x reference)
import jax
import jax.numpy as jnp
from jax.experimental import pallas as pl


def kernel(x_p_init, emb, c_init_w, c_init_b, cls_b, cl_wih, cl_whh, cl_bih, cl_bhh, lc_wih, lc_whh, lc_bih, lc_bhh, edge_index, p2c, c_t, p_t, y, num_iters):
    raise NotImplementedError("write your pallas kernel here")


# speedup vs baseline: 1.6204x; 1.6204x over previous
import functools

import jax
import jax.numpy as jnp
from jax import lax
from jax.experimental import pallas as pl
from jax.experimental.pallas import tpu as pltpu
from jax.experimental.pallas import tpu_sc as plsc

NL = 50000
NCL = 25000
D = 128
DH = 64
V = 400
E = 100000
EP = 102400
G = 128
NGRP = EP // G
NWORK = 32
RB = 1000
CB = 1000
F32 = jnp.float32
BF16 = jnp.bfloat16


def _sc_mesh():
    return plsc.VectorSubcoreMesh(core_axis_name="c", subcore_axis_name="s")


def _sc_gather(x_p, idx_r):
    @functools.partial(
        pl.kernel,
        out_type=jax.ShapeDtypeStruct((EP, D), F32),
        mesh=_sc_mesh(),
        scratch_types=[
            pltpu.VMEM((G,), jnp.int32),
            pltpu.VMEM((G, D), F32),
            pltpu.SemaphoreType.DMA,
        ],
    )
    def k(xp_hbm, idx_hbm, out_hbm, idx_v, rows_v, sem):
        wid = lax.axis_index("s") * 2 + lax.axis_index("c")

        def grp(j, carry):
            g = wid * (NGRP // NWORK) + j
            pltpu.sync_copy(idx_hbm.at[g], idx_v)
            pltpu.async_copy(xp_hbm.at[idx_v], rows_v, sem).wait()
            pltpu.sync_copy(rows_v, out_hbm.at[pl.ds(g * G, G)])
            return carry

        lax.fori_loop(0, NGRP // NWORK, grp, 0)

    return k(x_p, idx_r)


HALF0 = 12504
ACC = 12800
JUNK = ACC - 256


def _sc_scatter(x_c, src_r, dst2_r, zeros_acc):
    @functools.partial(
        pl.kernel,
        out_type=jax.ShapeDtypeStruct((NCL, D), F32),
        mesh=_sc_mesh(),
        scratch_types=[
            pltpu.VMEM((G,), jnp.int32),
            pltpu.VMEM((G,), jnp.int32),
            pltpu.VMEM((G, D), F32),
            pltpu.VMEM_SHARED((ACC, D), F32),
            pltpu.SemaphoreType.DMA,
        ],
    )
    def k(xc_hbm, src_hbm, dst_hbm, zero_hbm, out_hbm, src_v, dst_v, rows_v, acc, sem):
        c = lax.axis_index("c")
        s = lax.axis_index("s")
        rows_per_tile = ACC // 16
        base = s * rows_per_tile
        pltpu.sync_copy(zero_hbm.at[pl.ds(base, rows_per_tile)],
                        acc.at[pl.ds(base, rows_per_tile)])
        plsc.subcore_barrier()

        def grp(j, carry):
            g = s * (NGRP // 16) + j
            pltpu.sync_copy(src_hbm.at[g], src_v)
            pltpu.sync_copy(dst_hbm.at[c * NGRP + g], dst_v)
            pltpu.async_copy(xc_hbm.at[src_v], rows_v, sem).wait()
            pltpu.sync_copy(rows_v, acc.at[dst_v], add=True)
            return carry

        lax.fori_loop(0, NGRP // 16, grp, 0)
        plsc.subcore_barrier()

        sz = jnp.where(c == 0, HALF0, NCL - HALF0)
        lo = c * HALF0

        @pl.when(base + rows_per_tile <= sz)
        def _():
            pltpu.sync_copy(acc.at[pl.ds(base, rows_per_tile)],
                            out_hbm.at[pl.ds(lo + base, rows_per_tile)])

        t0 = HALF0 % rows_per_tile
        t1 = (NCL - HALF0) % rows_per_tile

        @pl.when(jnp.logical_and(c == 0,
                 jnp.logical_and(base < sz, base + rows_per_tile > sz)))
        def _():
            pltpu.sync_copy(acc.at[pl.ds(base, t0)],
                            out_hbm.at[pl.ds(lo + base, t0)])

        @pl.when(jnp.logical_and(c == 1,
                 jnp.logical_and(base < sz, base + rows_per_tile > sz)))
        def _():
            pltpu.sync_copy(acc.at[pl.ds(base, t1)],
                            out_hbm.at[pl.ds(lo + base, t1)])

    return k(x_c, src_r, dst2_r, zeros_acc)


def _init_body(y_ref, pt_ref, xpi_ref, emb_ref, o_ref):
    y = y_ref[...]
    oh = (y == lax.broadcasted_iota(jnp.int32, (1, V), 1)).astype(BF16)
    embs = jnp.dot(oh, emb_ref[...], preferred_element_type=F32)
    fixed = pt_ref[...] == 1
    o_ref[...] = jnp.where(fixed, embs, xpi_ref[...])


def _tc_init(y_r, pt_r, x_p_init, emb_bf):
    nb = NL // RB
    return pl.pallas_call(
        _init_body,
        grid=(nb,),
        in_specs=[
            pl.BlockSpec((RB, 1), lambda i: (i, 0)),
            pl.BlockSpec((RB, 1), lambda i: (i, 0)),
            pl.BlockSpec((RB, D), lambda i: (i, 0)),
            pl.BlockSpec((V, D), lambda i: (0, 0)),
        ],
        out_specs=pl.BlockSpec((RB, D), lambda i: (i, 0)),
        out_shape=jax.ShapeDtypeStruct((NL, D), F32),
    )(y_r, pt_r, x_p_init, emb_bf)


def _clause_body(vars_ref, xc_ref, xch_ref, ct_ref, wih_ref, whh_ref, b_ref,
                 h_ref, c_ref):
    v = vars_ref[...].astype(BF16)
    hprev = xc_ref[...]
    hb = hprev.astype(BF16)
    ct = ct_ref[...]
    gates = jnp.zeros((CB, 4 * D), F32)
    for k in range(4):
        gk = (jnp.dot(v, wih_ref[k], preferred_element_type=F32)
              + jnp.dot(hb, whh_ref[k], preferred_element_type=F32)
              + b_ref[k][None, :])
        gates = jnp.where(ct == k, gk, gates)
    i_, f_, g_, o_ = jnp.split(gates, 4, axis=-1)
    c_new = jax.nn.sigmoid(f_) * xch_ref[...] + jax.nn.sigmoid(i_) * jnp.tanh(g_)
    h_new = jax.nn.sigmoid(o_) * jnp.tanh(c_new)
    h_ref[...] = h_new
    c_ref[...] = c_new


def _tc_clause(vars2d, x_c, x_ch, ct_r, wih_t, whh_t, lcb):
    nb = NCL // CB
    return pl.pallas_call(
        _clause_body,
        grid=(nb,),
        in_specs=[
            pl.BlockSpec((CB, 4 * D), lambda i: (i, 0)),
            pl.BlockSpec((CB, D), lambda i: (i, 0)),
            pl.BlockSpec((CB, D), lambda i: (i, 0)),
            pl.BlockSpec((CB, 1), lambda i: (i, 0)),
            pl.BlockSpec((4, 4 * D, 4 * D), lambda i: (0, 0, 0)),
            pl.BlockSpec((4, D, 4 * D), lambda i: (0, 0, 0)),
            pl.BlockSpec((4, 4 * D), lambda i: (0, 0)),
        ],
        out_specs=[
            pl.BlockSpec((CB, D), lambda i: (i, 0)),
            pl.BlockSpec((CB, D), lambda i: (i, 0)),
        ],
        out_shape=[
            jax.ShapeDtypeStruct((NCL, D), F32),
            jax.ShapeDtypeStruct((NCL, D), F32),
        ],
    )(vars2d, x_c, x_ch, ct_r, wih_t, whh_t, lcb)


def _lit_body(msg_ref, xp_ref, xph_ref, pt_ref, wih_ref, whh_ref,
              b_ref, ho_ref, co_ref):
    i = pl.program_id(0)
    has_msg = (i < NCL // RB).astype(F32)
    xp = xp_ref[...]
    xph = xph_ref[...]
    gates = (jnp.dot(xp.astype(BF16), whh_ref[...], preferred_element_type=F32)
             + b_ref[0][None, :])
    msg_g = jnp.dot(msg_ref[...].astype(BF16), wih_ref[...],
                    preferred_element_type=F32)
    gates = gates + has_msg * msg_g
    i_, f_, g_, o_ = jnp.split(gates, 4, axis=-1)
    c_new = jax.nn.sigmoid(f_) * xph + jax.nn.sigmoid(i_) * jnp.tanh(g_)
    h_new = jax.nn.sigmoid(o_) * jnp.tanh(c_new)
    var = pt_ref[...] == 0
    ho_ref[...] = jnp.where(var, h_new, xp)
    co_ref[...] = jnp.where(var, c_new, xph)


def _tc_lit(msg, x_p, x_ph, pt_r, wih_t, whh_t, clb):
    nb = NL // RB
    nmb = NCL // RB
    return pl.pallas_call(
        _lit_body,
        grid=(nb,),
        in_specs=[
            pl.BlockSpec((RB, D), lambda i: (jnp.minimum(i, nmb - 1), 0)),
            pl.BlockSpec((RB, D), lambda i: (i, 0)),
            pl.BlockSpec((RB, D), lambda i: (i, 0)),
            pl.BlockSpec((RB, 1), lambda i: (i, 0)),
            pl.BlockSpec((D, 4 * D), lambda i: (0, 0)),
            pl.BlockSpec((D, 4 * D), lambda i: (0, 0)),
            pl.BlockSpec((1, 4 * D), lambda i: (0, 0)),
        ],
        out_specs=[
            pl.BlockSpec((RB, D), lambda i: (i, 0)),
            pl.BlockSpec((RB, D), lambda i: (i, 0)),
        ],
        out_shape=[
            jax.ShapeDtypeStruct((NL, D), F32),
            jax.ShapeDtypeStruct((NL, D), F32),
        ],
    )(msg, x_p, x_ph, pt_r, wih_t, whh_t, clb)


def _cls_body(xp_ref, embt_ref, b_ref, o_ref):
    o_ref[...] = (jnp.dot(xp_ref[...].astype(BF16), embt_ref[...],
                          preferred_element_type=F32) + b_ref[0][None, :])


def _tc_cls(x_p, emb_t_pad, clsb_pad):
    nb = NL // RB
    vp = emb_t_pad.shape[1]
    return pl.pallas_call(
        _cls_body,
        grid=(nb,),
        in_specs=[
            pl.BlockSpec((RB, D), lambda i: (i, 0)),
            pl.BlockSpec((D, vp), lambda i: (0, 0)),
            pl.BlockSpec((1, vp), lambda i: (0, 0)),
        ],
        out_specs=pl.BlockSpec((RB, vp), lambda i: (i, 0)),
        out_shape=jax.ShapeDtypeStruct((NL, vp), F32),
    )(x_p, emb_t_pad, clsb_pad)


def kernel(x_p_init, emb, c_init_w, c_init_b, cls_b, cl_wih, cl_whh, cl_bih,
           cl_bhh, lc_wih, lc_whh, lc_bih, lc_bhh, edge_index, p2c, c_t, p_t,
           y, num_iters):
    pad = EP - E
    p2c_r = jnp.concatenate([p2c, jnp.zeros((pad,), jnp.int32)]).reshape(NGRP, G)
    src = edge_index[0]
    dst = edge_index[1]
    src_r = jnp.concatenate([src, jnp.zeros((pad,), jnp.int32)]).reshape(NGRP, G)
    dst_pad = jnp.concatenate([dst, jnp.full((pad,), NCL, jnp.int32)])
    dst_cores = []
    for c in range(2):
        lo, hi = (0, HALF0) if c == 0 else (HALF0, NCL)
        in_range = jnp.logical_and(dst_pad >= lo, dst_pad < hi)
        dst_cores.append(jnp.where(in_range, dst_pad - lo,
                                   JUNK + (dst_pad & 255)))
    dst2_r = jnp.stack(dst_cores).reshape(2 * NGRP, G)
    zeros_acc = jnp.zeros((ACC, D), F32)
    wih_t = jnp.transpose(lc_wih, (0, 2, 1))
    wih_t = wih_t.at[3, 3 * D:, :].set(0.0).astype(BF16)
    whh_t = jnp.transpose(lc_whh, (0, 2, 1)).astype(BF16)
    lcb = lc_bih + lc_bhh
    cl_wih_t = cl_wih.T.astype(BF16)
    cl_whh_t = cl_whh.T.astype(BF16)
    clb = (cl_bih + cl_bhh).reshape(1, 4 * D)
    emb_bf = emb.astype(BF16)
    vpad = 512
    emb_t_pad = jnp.pad(emb, ((0, vpad - V), (0, 0))).T.astype(BF16)
    clsb_pad = jnp.pad(cls_b, (0, vpad - V)).reshape(1, vpad)
    y_r = y.reshape(NL, 1)
    pt_r = p_t.reshape(NL, 1)
    ct_r = c_t.reshape(NCL, 1)

    x_p = _tc_init(y_r, pt_r, x_p_init, emb_bf)
    x_ph = jnp.zeros((NL, D), F32)
    c0 = c_init_w[:, 0] + c_init_b
    x_c = jnp.broadcast_to(c0[None, :], (NCL, D))
    x_ch = jnp.zeros((NCL, D), F32)

    def body(_, carry):
        x_p, x_ph, x_c, x_ch = carry
        vars2d = _sc_gather(x_p, p2c_r).reshape(EP // 4, 4 * D)
        x_c, x_ch = _tc_clause(vars2d, x_c, x_ch, ct_r, wih_t, whh_t, lcb)
        msg = _sc_scatter(x_c, src_r, dst2_r, zeros_acc)
        x_p, x_ph = _tc_lit(msg, x_p, x_ph, pt_r, cl_wih_t, cl_whh_t, clb)
        return (x_p, x_ph, x_c, x_ch)

    x_p, x_ph, x_c, x_ch = lax.fori_loop(0, num_iters, body,
                                         (x_p, x_ph, x_c, x_ch))
    logits_pad = _tc_cls(x_p, emb_t_pad, clsb_pad)
    return logits_pad[:, :V]
